# baseline (speedup 1.0000x reference)
import functools

import jax
import jax.numpy as jnp
from jax import lax
from jax.experimental import pallas as pl
from jax.experimental.pallas import tpu as pltpu
from jax.experimental.pallas import tpu_sc as plsc

D = 64
NC = 2
NS = 16
NW = NC * NS
L = 16
BBLK = 128
DT = D // 8
TS = D + 1


def _emb_grid(n_b, n_g, n_tab):
    assert n_b == NW * BBLK
    assert n_g % 2 == 0

    mesh = plsc.VectorSubcoreMesh(core_axis_name="c", subcore_axis_name="s")

    @functools.partial(
        pl.kernel,
        mesh=mesh,
        out_type=jax.ShapeDtypeStruct((n_g, DT, NW, 8, BBLK), jnp.float32),
        scratch_types=[
            pltpu.VMEM((n_g, BBLK), jnp.int32),
            pltpu.VMEM((n_tab,), jnp.float32),
            pltpu.VMEM((2, DT, 8, BBLK), jnp.float32),
            pltpu.VMEM((2, DT, 8, BBLK), jnp.float32),
            pltpu.SemaphoreType.DMA,
            pltpu.SemaphoreType.DMA,
        ],
        compiler_params=pltpu.CompilerParams(
            use_tc_tiling_on_sc=False,
            needs_layout_passes=False,
            disable_bounds_checks=True,
        ),
    )
    def emb(expr_hbm, tab_hbm, out_hbm, idx_v, tab_v, m0, m1, so0, so1):
        wid = lax.axis_index("s") * NC + lax.axis_index("c")
        b0 = wid * BBLK

        pltpu.sync_copy(expr_hbm.at[:, pl.ds(b0, BBLK)], idx_v)
        pltpu.sync_copy(tab_hbm, tab_v)

        def scale(r, _):
            for j in range(BBLK // L):
                sl = pl.ds(j * L, L)
                idx_v[r, sl] = idx_v[r, sl] * TS
            return 0

        lax.fori_loop(0, n_g, scale, 0)

        def fill(g, m_ref, h):
            for j in range(BBLK // L):
                base = idx_v[g, pl.ds(j * L, L)]

                @plsc.parallel_loop(0, D // L, unroll=4)
                def _grp(grp):
                    bvec = base + grp * L
                    vals = [
                        plsc.load_gather(tab_v, [bvec + k]) for k in range(L)
                    ]
                    for k in range(L):
                        m_ref[h, 2 * grp + k // 8, k % 8, pl.ds(j * L, L)] = (
                            vals[k]
                        )

        def fill2(g, m_ref):
            fill(g, m_ref, 0)
            fill(g + 1, m_ref, 1)

        def fire_out(g, m_ref, sem):
            pltpu.async_copy(m_ref, out_hbm.at[pl.ds(g, 2), :, wid, :, :], sem)

        def wait_out(g, m_ref, sem):
            pltpu.make_async_copy(
                m_ref, out_hbm.at[pl.ds(g, 2), :, wid, :, :], sem
            ).wait()

        fill2(0, m0)
        fire_out(0, m0, so0)
        fill2(2, m1)
        fire_out(2, m1, so1)

        def body(t, _):
            g0 = 4 * t
            wait_out(g0 - 4, m0, so0)
            fill2(g0, m0)
            fire_out(g0, m0, so0)
            wait_out(g0 - 2, m1, so1)
            fill2(g0 + 2, m1)
            fire_out(g0 + 2, m1, so1)
            return 0

        lax.fori_loop(1, n_g // 4, body, 0)

        wait_out(n_g - 4, m0, so0)
        wait_out(n_g - 2, m1, so1)

    return emb


def kernel(expression, table):
    b, g = expression.shape
    v, d = table.shape
    expr_t = expression.T
    tab_flat = jnp.pad(table, ((0, 0), (0, TS - d))).reshape(v * TS)
    out5 = _emb_grid(b, g, v * TS)(expr_t, tab_flat)
    out = jnp.transpose(out5, (2, 4, 0, 1, 3)).reshape(b, g, d)
    return out


# speedup vs baseline: 1.0100x; 1.0100x over previous
import functools

import jax
import jax.numpy as jnp
from jax import lax
from jax.experimental import pallas as pl
from jax.experimental.pallas import tpu as pltpu
from jax.experimental.pallas import tpu_sc as plsc

D = 64
NC = 2
NS = 16
NW = NC * NS
L = 16
BBLK = 128
DT = D // 8
TS = D + 1


def _emb_grid(n_b, n_g, n_tab):
    assert n_b == NW * BBLK
    assert n_g % 2 == 0

    mesh = plsc.VectorSubcoreMesh(core_axis_name="c", subcore_axis_name="s")

    @functools.partial(
        pl.kernel,
        mesh=mesh,
        out_type=jax.ShapeDtypeStruct((n_g, DT, NW, 8, BBLK), jnp.float32),
        scratch_types=[
            pltpu.VMEM((n_g, BBLK), jnp.int32),
            pltpu.VMEM((n_tab,), jnp.float32),
            pltpu.VMEM((DT, 8, BBLK), jnp.float32),
            pltpu.VMEM((DT, 8, BBLK), jnp.float32),
            pltpu.VMEM((DT, 8, BBLK), jnp.float32),
            pltpu.VMEM((DT, 8, BBLK), jnp.float32),
            pltpu.SemaphoreType.DMA,
            pltpu.SemaphoreType.DMA,
            pltpu.SemaphoreType.DMA,
            pltpu.SemaphoreType.DMA,
        ],
        compiler_params=pltpu.CompilerParams(
            use_tc_tiling_on_sc=False,
            needs_layout_passes=False,
            disable_bounds_checks=True,
        ),
    )
    def emb(
        expr_hbm, tab_hbm, out_hbm, idx_v, tab_v,
        m0, m1, m2, m3, so0, so1, so2, so3,
    ):
        wid = lax.axis_index("s") * NC + lax.axis_index("c")
        b0 = wid * BBLK

        pltpu.sync_copy(expr_hbm.at[:, pl.ds(b0, BBLK)], idx_v)
        pltpu.sync_copy(tab_hbm, tab_v)

        def scale(r, _):
            for j in range(BBLK // L):
                sl = pl.ds(j * L, L)
                idx_v[r, sl] = idx_v[r, sl] * TS
            return 0

        lax.fori_loop(0, n_g, scale, 0)

        def fill(g, m_ref):
            for j in range(BBLK // L):
                base = idx_v[g, pl.ds(j * L, L)]

                @plsc.parallel_loop(0, D // L, unroll=4)
                def _grp(grp):
                    bvec = base + grp * L
                    vals = [
                        plsc.load_gather(tab_v, [bvec + k]) for k in range(L)
                    ]
                    for k in range(L):
                        m_ref[2 * grp + k // 8, k % 8, pl.ds(j * L, L)] = (
                            vals[k]
                        )

        def fire_out(g, m_ref, sem):
            pltpu.async_copy(m_ref, out_hbm.at[g, :, wid, :, :], sem)

        def wait_out(g, m_ref, sem):
            pltpu.make_async_copy(m_ref, out_hbm.at[g, :, wid, :, :], sem).wait()

        bufs = ((m0, so0), (m1, so1), (m2, so2), (m3, so3))
        for c, (m, so) in enumerate(bufs):
            fill(c, m)
            fire_out(c, m, so)

        def body(t, _):
            g0 = 4 * t
            for c, (m, so) in enumerate(bufs):
                wait_out(g0 + c - 4, m, so)
                fill(g0 + c, m)
                fire_out(g0 + c, m, so)
            return 0

        lax.fori_loop(1, n_g // 4, body, 0)

        for c, (m, so) in enumerate(bufs):
            wait_out(n_g - 4 + c, m, so)

    return emb


def kernel(expression, table):
    b, g = expression.shape
    v, d = table.shape
    expr_t = expression.T
    tab_flat = jnp.pad(table, ((0, 0), (0, TS - d))).reshape(v * TS)
    out5 = _emb_grid(b, g, v * TS)(expr_t, tab_flat)
    out = jnp.transpose(out5, (2, 4, 0, 1, 3)).reshape(b, g, d)
    return out


# speedup vs baseline: 2.2428x; 2.2207x over previous
import functools

import jax
import jax.numpy as jnp
from jax import lax
from jax.experimental import pallas as pl
from jax.experimental.pallas import tpu as pltpu
from jax.experimental.pallas import tpu_sc as plsc

D = 64
NC = 2
NS = 16
NW = NC * NS
L = 16
BBLK = 128
DT = D // 8
TS = D + 1


def _emb_grid(n_b, n_g, n_tab):
    assert n_b == NW * BBLK
    assert n_g % 2 == 0

    mesh = plsc.VectorSubcoreMesh(core_axis_name="c", subcore_axis_name="s")

    @functools.partial(
        pl.kernel,
        mesh=mesh,
        out_type=jax.ShapeDtypeStruct((n_g, DT, NW, 8, BBLK), jnp.float32),
        scratch_types=[
            pltpu.VMEM((n_g, BBLK), jnp.int32),
            pltpu.VMEM((n_tab,), jnp.float32),
            pltpu.VMEM((DT, 8, BBLK), jnp.float32),
            pltpu.VMEM((DT, 8, BBLK), jnp.float32),
            pltpu.SemaphoreType.DMA,
            pltpu.SemaphoreType.DMA,
        ],
        compiler_params=pltpu.CompilerParams(
            use_tc_tiling_on_sc=False,
            needs_layout_passes=False,
            disable_bounds_checks=True,
        ),
    )
    def emb(expr_hbm, tab_hbm, out_hbm, idx_v, tab_v, m0, m1, so0, so1):
        wid = lax.axis_index("s") * NC + lax.axis_index("c")
        b0 = wid * BBLK

        pltpu.sync_copy(expr_hbm.at[:, pl.ds(b0, BBLK)], idx_v)
        pltpu.sync_copy(tab_hbm, tab_v)

        @plsc.parallel_loop(0, n_g, unroll=4)
        def _scale(r):
            for j in range(BBLK // L):
                sl = pl.ds(j * L, L)
                idx_v[r, sl] = idx_v[r, sl] * TS

        def fill(g, m_ref):
            bases = [idx_v[g, pl.ds(j * L, L)] for j in range(BBLK // L)]
            for j in range(BBLK // L):
                base = bases[j]

                @plsc.parallel_loop(0, D // L, unroll=4)
                def _grp(grp):
                    bvec = base + grp * L
                    vals = [
                        plsc.load_gather(tab_v, [bvec + k]) for k in range(L)
                    ]
                    for k in range(L):
                        m_ref[2 * grp + k // 8, k % 8, pl.ds(j * L, L)] = (
                            vals[k]
                        )

        def fire_out(g, m_ref, sem):
            pltpu.async_copy(m_ref, out_hbm.at[g, :, wid, :, :], sem)

        def wait_out(g, m_ref, sem):
            pltpu.make_async_copy(m_ref, out_hbm.at[g, :, wid, :, :], sem).wait()

        fill(0, m0)
        fire_out(0, m0, so0)
        fill(1, m1)
        fire_out(1, m1, so1)

        def body(t, _):
            g0 = 2 * t
            g1 = g0 + 1
            wait_out(g0 - 2, m0, so0)
            fill(g0, m0)
            fire_out(g0, m0, so0)
            wait_out(g1 - 2, m1, so1)
            fill(g1, m1)
            fire_out(g1, m1, so1)
            return 0

        lax.fori_loop(1, n_g // 2, body, 0)

        wait_out(n_g - 2, m0, so0)
        wait_out(n_g - 1, m1, so1)

    return emb


def kernel(expression, table):
    b, g = expression.shape
    v, d = table.shape
    expr_t = expression.T
    tab_flat = jnp.pad(table, ((0, 0), (0, TS - d))).reshape(v * TS)
    out5 = _emb_grid(b, g, v * TS)(expr_t, tab_flat)
    out = jnp.transpose(out5, (2, 4, 0, 1, 3)).reshape(b, g, d)
    return out


# speedup vs baseline: 2.2504x; 1.0034x over previous
import functools

import jax
import jax.numpy as jnp
from jax import lax
from jax.experimental import pallas as pl
from jax.experimental.pallas import tpu as pltpu
from jax.experimental.pallas import tpu_sc as plsc

D = 64
NC = 2
NS = 16
NW = NC * NS
L = 16
BBLK = 128
DT = D // 8
TS = D + 1


def _emb_grid(n_b, n_g, n_tab):
    assert n_b == NW * BBLK
    assert n_g % 2 == 0

    mesh = plsc.VectorSubcoreMesh(core_axis_name="c", subcore_axis_name="s")

    @functools.partial(
        pl.kernel,
        mesh=mesh,
        out_type=jax.ShapeDtypeStruct((n_g, DT, NW, 8, BBLK), jnp.float32),
        scratch_types=[
            pltpu.VMEM((n_g, BBLK), jnp.int32),
            pltpu.VMEM((n_tab,), jnp.float32),
            pltpu.VMEM((DT, 8, BBLK), jnp.float32),
            pltpu.VMEM((DT, 8, BBLK), jnp.float32),
            pltpu.SemaphoreType.DMA,
            pltpu.SemaphoreType.DMA,
        ],
        compiler_params=pltpu.CompilerParams(
            use_tc_tiling_on_sc=False,
            needs_layout_passes=False,
            disable_bounds_checks=True,
        ),
    )
    def emb(expr_hbm, tab_hbm, out_hbm, idx_v, tab_v, m0, m1, so0, so1):
        wid = lax.axis_index("s") * NC + lax.axis_index("c")
        b0 = wid * BBLK

        c_idx = pltpu.async_copy(expr_hbm.at[:, pl.ds(b0, BBLK)], idx_v, so0)
        c_tab = pltpu.async_copy(tab_hbm, tab_v, so1)
        c_idx.wait()
        c_tab.wait()

        @plsc.parallel_loop(0, n_g, unroll=4)
        def _scale(r):
            for j in range(BBLK // L):
                sl = pl.ds(j * L, L)
                idx_v[r, sl] = idx_v[r, sl] * TS

        def fill(g, m_ref):
            bases = [idx_v[g, pl.ds(j * L, L)] for j in range(BBLK // L)]
            for j in range(BBLK // L):
                base = bases[j]

                @plsc.parallel_loop(0, D // L, unroll=4)
                def _grp(grp):
                    bvec = base + grp * L
                    vals = [
                        plsc.load_gather(tab_v, [bvec + k]) for k in range(L)
                    ]
                    for k in range(L):
                        m_ref[2 * grp + k // 8, k % 8, pl.ds(j * L, L)] = (
                            vals[k]
                        )

        def fire_out(g, m_ref, sem):
            pltpu.async_copy(m_ref, out_hbm.at[g, :, wid, :, :], sem)

        def wait_out(g, m_ref, sem):
            pltpu.make_async_copy(m_ref, out_hbm.at[g, :, wid, :, :], sem).wait()

        fill(0, m0)
        fire_out(0, m0, so0)
        fill(1, m1)
        fire_out(1, m1, so1)

        def body(t, _):
            g0 = 2 * t
            g1 = g0 + 1
            wait_out(g0 - 2, m0, so0)
            fill(g0, m0)
            fire_out(g0, m0, so0)
            wait_out(g1 - 2, m1, so1)
            fill(g1, m1)
            fire_out(g1, m1, so1)
            return 0

        lax.fori_loop(1, n_g // 2, body, 0)

        wait_out(n_g - 2, m0, so0)
        wait_out(n_g - 1, m1, so1)

    return emb


def kernel(expression, table):
    b, g = expression.shape
    v, d = table.shape
    expr_t = expression.T
    tab_flat = jnp.pad(table, ((0, 0), (0, TS - d))).reshape(v * TS)
    out5 = _emb_grid(b, g, v * TS)(expr_t, tab_flat)
    out = jnp.transpose(out5, (2, 4, 0, 1, 3)).reshape(b, g, d)
    return out
